# baseline (speedup 1.0000x reference)
import numpy as np
import jax
import jax.numpy as jnp
from jax.experimental import pallas as pl
from jax.experimental.pallas import tpu as pltpu

_FEATURE_MAPS = [(80, 80), (40, 40), (20, 20)]
_INPUT_SIZE = 640.0
_NUM_CLASSES = 80
_TOPK = 10
_CLS_FACTOR = 0.5
_IOU_FACTOR = 6.0
_EPS = 1e-9


def _anchors_np():
    all_a, all_s = [], []
    for (h, w) in _FEATURE_MAPS:
        stride = _INPUT_SIZE / float(h)
        ys = (np.arange(h, dtype=np.float32) + 0.5) * stride
        xs = (np.arange(w, dtype=np.float32) + 0.5) * stride
        gx, gy = np.meshgrid(xs, ys)
        all_a.append(np.stack([gx.reshape(-1), gy.reshape(-1)], axis=-1))
        all_s.append(np.full((h * w,), stride, dtype=np.float32))
    anchors = np.concatenate(all_a, axis=0).astype(np.float32)
    scalers = np.concatenate(all_s, axis=0).astype(np.float32)
    return anchors, scalers


def _yolo_kernel(cls_ref, boxes_ref, boxes_t_ref, tc_ref, tb_ref,
                 anc_t_ref, inv_s_col_ref, inv_s_row_ref, out_ref):
    A = cls_ref.shape[1]
    T = tb_ref.shape[1]
    C = cls_ref.shape[2]
    f32 = jnp.float32

    cls = cls_ref[0]
    boxes = boxes_ref[0]
    boxes_t = boxes_t_ref[0]
    tcls = tc_ref[0]
    tb = tb_ref[0]
    anc_t = anc_t_ref[...]
    inv_s_col = inv_s_col_ref[...]
    inv_s_row = inv_s_row_ref[...]

    iota_c = jax.lax.broadcasted_iota(jnp.int32, (T, C), 1)
    onehot_tc = (iota_c == tcls).astype(f32)
    bf16 = jnp.bfloat16
    oh_b = onehot_tc.astype(bf16)
    cls_t = jnp.transpose(cls)
    p0 = cls_t.astype(bf16)
    r1 = cls_t - p0.astype(f32)
    p1 = r1.astype(bf16)
    p2 = (r1 - p1.astype(f32)).astype(bf16)

    def _csdot(rhs_b):
        return jax.lax.dot_general(
            oh_b, rhs_b, (((1,), (0,)), ((), ())),
            preferred_element_type=f32)

    logit = _csdot(p0) + _csdot(p1) + _csdot(p2)
    sqrt_cs = jax.lax.rsqrt(1.0 + jnp.exp(-logit))

    gx1 = tb[:, 0:1]; gy1 = tb[:, 1:2]; gx2 = tb[:, 2:3]; gy2 = tb[:, 3:4]
    px1 = boxes_t[0:1, :]; py1 = boxes_t[1:2, :]
    px2 = boxes_t[2:3, :]; py2 = boxes_t[3:4, :]
    iw = jnp.maximum(jnp.minimum(gx2, px2) - jnp.maximum(gx1, px1), 0.0)
    ih = jnp.maximum(jnp.minimum(gy2, py2) - jnp.maximum(gy1, py1), 0.0)
    inter = iw * ih
    ag = (gx2 - gx1) * (gy2 - gy1)
    ap = (px2 - px1) * (py2 - py1)
    iou = inter / (ag + ap - inter + _EPS)

    iou2 = iou * iou
    metric = sqrt_cs * (iou2 * iou2 * iou2)

    ax = anc_t[0:1, :]; ay = anc_t[1:2, :]
    in_gt = (ax >= gx1) & (ax <= gx2) & (ay >= gy1) & (ay <= gy2)
    metric_m = jnp.where(in_gt, metric, 0.0)

    W = 128
    NC = (A + W - 1) // W
    pad = NC * W - A
    bufs = [jnp.full((T, W), -1.0, f32)] * _TOPK
    for i in range(NC):
        if pad and i == NC - 1:
            c = jnp.concatenate(
                [metric_m[:, i * W:], jnp.zeros((T, pad), f32)], axis=1)
        else:
            c = metric_m[:, i * W:(i + 1) * W]
        for j in range(min(i + 1, _TOPK)):
            hi = jnp.maximum(bufs[j], c)
            if j != _TOPK - 1:
                c = jnp.minimum(bufs[j], c)
            bufs[j] = hi
    buf = jnp.concatenate(bufs, axis=1)
    total = jnp.zeros((T, 1), f32)
    kth = jnp.zeros((T, 1), f32)
    for r in range(_TOPK):
        mx = jnp.max(buf, axis=1, keepdims=True)
        eq = buf == mx
        c = jnp.sum(jnp.where(eq, 1.0, 0.0), axis=1, keepdims=True)
        hit = (total < float(_TOPK)) & (total + c >= float(_TOPK))
        kth = jnp.where(hit, mx, kth)
        total = total + c
        if r != _TOPK - 1:
            buf = jnp.where(eq, -1.0, buf)

    mask_pos = (metric_m >= kth) & (metric_m > 0.0)
    mask_pos_f = mask_pos.astype(f32)
    n_assign = jnp.sum(mask_pos_f, axis=0, keepdims=True)

    iou_m = jnp.where(mask_pos, iou, -1.0)
    mxi = jnp.max(iou_m, axis=0, keepdims=True)
    iota_t = jax.lax.broadcasted_iota(jnp.int32, (T, A), 0)
    cand_t = jnp.where(iou_m == mxi, iota_t, T)
    first_t = jnp.min(cand_t, axis=0, keepdims=True)
    is_max = (iota_t == first_t).astype(f32)
    mask_pos_f = jnp.where(n_assign > 1.0, is_max, mask_pos_f)

    metric_pos = metric_m * mask_pos_f
    iou_pos = jnp.where(mask_pos_f > 0.0, iou, 0.0)
    pmm = jnp.max(metric_pos, axis=1, keepdims=True)
    pim = jnp.max(iou_pos, axis=1, keepdims=True)
    q = metric_pos * (pim / (pmm + _EPS))
    q = jnp.maximum(q, mask_pos_f * 1e-30)

    rhs = jnp.concatenate(
        [tb, jnp.ones((T, 1), f32), onehot_tc], axis=1)
    lhs_b = q.astype(bf16)
    rhs0 = rhs.astype(bf16)
    rhs1 = (rhs - rhs0.astype(f32)).astype(bf16)

    def _pkdot(rhs_b):
        return jax.lax.dot_general(
            lhs_b, rhs_b, (((0,), (0,)), ((), ())),
            preferred_element_type=f32)

    packed = _pkdot(rhs0) + _pkdot(rhs1)

    coef_col = packed[:, 4:5]
    fg = coef_col > 0.0
    fg_col = fg.astype(f32)
    denom = jnp.maximum(coef_col, 1e-38)
    bbox_n = jnp.where(fg, packed[:, 0:4] / denom, tb[0:1, :]) * inv_s_col
    align_cls = packed[:, 5:]

    boxes_n = boxes * inv_s_col
    out_ref[0, :, 0:4] = boxes_n
    out_ref[0, :, 4:8] = bbox_n
    out_ref[0, :, 8:9] = fg_col
    out_ref[0, :, 9:] = align_cls


def kernel(cls, boxes, true_classes, true_bboxes):
    B, A, C = cls.shape
    T = true_classes.shape[1]
    anchors, scalers = _anchors_np()
    anc_t = jnp.asarray(anchors.T)
    inv_s = (1.0 / scalers).astype(np.float32)
    inv_s_col = jnp.asarray(inv_s[:, None])
    inv_s_row = jnp.asarray(inv_s[None, :])

    boxes_t = jnp.transpose(boxes, (0, 2, 1))
    tc = true_classes.astype(jnp.int32)[..., None]

    out = pl.pallas_call(
        _yolo_kernel,
        grid=(B,),
        in_specs=[
            pl.BlockSpec((1, A, C), lambda b: (b, 0, 0)),
            pl.BlockSpec((1, A, 4), lambda b: (b, 0, 0)),
            pl.BlockSpec((1, 4, A), lambda b: (b, 0, 0)),
            pl.BlockSpec((1, T, 1), lambda b: (b, 0, 0)),
            pl.BlockSpec((1, T, 4), lambda b: (b, 0, 0)),
            pl.BlockSpec((2, A), lambda b: (0, 0)),
            pl.BlockSpec((A, 1), lambda b: (0, 0)),
            pl.BlockSpec((1, A), lambda b: (0, 0)),
        ],
        out_specs=pl.BlockSpec((1, A, 4 + 4 + 1 + C), lambda b: (b, 0, 0)),
        out_shape=jax.ShapeDtypeStruct((B, A, 4 + 4 + 1 + C), cls.dtype),
        compiler_params=pltpu.CompilerParams(
            dimension_semantics=("arbitrary",),
            vmem_limit_bytes=100 * 1024 * 1024,
        ),
    )(cls, boxes, boxes_t, tc, true_bboxes, anc_t, inv_s_col, inv_s_row)
    return out


# speedup vs baseline: 1.0113x; 1.0113x over previous
import numpy as np
import jax
import jax.numpy as jnp
from jax.experimental import pallas as pl
from jax.experimental.pallas import tpu as pltpu

_FEATURE_MAPS = [(80, 80), (40, 40), (20, 20)]
_INPUT_SIZE = 640.0
_NUM_CLASSES = 80
_TOPK = 10
_CLS_FACTOR = 0.5
_IOU_FACTOR = 6.0
_EPS = 1e-9


def _anchors_np():
    all_a, all_s = [], []
    for (h, w) in _FEATURE_MAPS:
        stride = _INPUT_SIZE / float(h)
        ys = (np.arange(h, dtype=np.float32) + 0.5) * stride
        xs = (np.arange(w, dtype=np.float32) + 0.5) * stride
        gx, gy = np.meshgrid(xs, ys)
        all_a.append(np.stack([gx.reshape(-1), gy.reshape(-1)], axis=-1))
        all_s.append(np.full((h * w,), stride, dtype=np.float32))
    anchors = np.concatenate(all_a, axis=0).astype(np.float32)
    scalers = np.concatenate(all_s, axis=0).astype(np.float32)
    return anchors, scalers


def _yolo_kernel(cls_ref, boxes_ref, boxes_t_ref, tc_ref, tb_ref,
                 anc_t_ref, inv_s_col_ref, inv_s_row_ref, out_ref):
    A = cls_ref.shape[1]
    T = tb_ref.shape[1]
    C = cls_ref.shape[2]
    f32 = jnp.float32

    cls = cls_ref[0]
    boxes = boxes_ref[0]
    boxes_t = boxes_t_ref[0]
    tcls = tc_ref[0]
    tb = tb_ref[0]
    anc_t = anc_t_ref[...]
    inv_s_col = inv_s_col_ref[...]
    inv_s_row = inv_s_row_ref[...]

    iota_c = jax.lax.broadcasted_iota(jnp.int32, (T, C), 1)
    onehot_tc = (iota_c == tcls).astype(f32)
    bf16 = jnp.bfloat16
    oh_b = onehot_tc.astype(bf16)
    p0 = cls.astype(bf16)
    r1 = cls - p0.astype(f32)
    p1 = r1.astype(bf16)
    p2 = (r1 - p1.astype(f32)).astype(bf16)

    def _csdot(rhs_b):
        return jax.lax.dot_general(
            oh_b, rhs_b, (((1,), (1,)), ((), ())),
            preferred_element_type=f32)

    logit = _csdot(p0) + _csdot(p1) + _csdot(p2)
    sqrt_cs = jax.lax.rsqrt(1.0 + jnp.exp(-logit))

    gx1 = tb[:, 0:1]; gy1 = tb[:, 1:2]; gx2 = tb[:, 2:3]; gy2 = tb[:, 3:4]
    px1 = boxes_t[0:1, :]; py1 = boxes_t[1:2, :]
    px2 = boxes_t[2:3, :]; py2 = boxes_t[3:4, :]
    iw = jnp.maximum(jnp.minimum(gx2, px2) - jnp.maximum(gx1, px1), 0.0)
    ih = jnp.maximum(jnp.minimum(gy2, py2) - jnp.maximum(gy1, py1), 0.0)
    inter = iw * ih
    ag = (gx2 - gx1) * (gy2 - gy1)
    ap = (px2 - px1) * (py2 - py1)
    iou = inter / (ag + ap - inter + _EPS)

    iou2 = iou * iou
    metric = sqrt_cs * (iou2 * iou2 * iou2)

    ax = anc_t[0:1, :]; ay = anc_t[1:2, :]
    in_gt = (ax >= gx1) & (ax <= gx2) & (ay >= gy1) & (ay <= gy2)
    metric_m = jnp.where(in_gt, metric, 0.0)

    W = 128
    NC = (A + W - 1) // W
    pad = NC * W - A
    bufs = [jnp.full((T, W), -1.0, f32)] * _TOPK
    for i in range(NC):
        if pad and i == NC - 1:
            c = jnp.concatenate(
                [metric_m[:, i * W:], jnp.zeros((T, pad), f32)], axis=1)
        else:
            c = metric_m[:, i * W:(i + 1) * W]
        for j in range(min(i + 1, _TOPK)):
            hi = jnp.maximum(bufs[j], c)
            if j != _TOPK - 1:
                c = jnp.minimum(bufs[j], c)
            bufs[j] = hi
    buf = jnp.concatenate(bufs, axis=1)
    total = jnp.zeros((T, 1), f32)
    kth = jnp.zeros((T, 1), f32)
    for r in range(_TOPK):
        mx = jnp.max(buf, axis=1, keepdims=True)
        eq = buf == mx
        c = jnp.sum(jnp.where(eq, 1.0, 0.0), axis=1, keepdims=True)
        hit = (total < float(_TOPK)) & (total + c >= float(_TOPK))
        kth = jnp.where(hit, mx, kth)
        total = total + c
        if r != _TOPK - 1:
            buf = jnp.where(eq, -1.0, buf)

    mask_pos = (metric_m >= kth) & (metric_m > 0.0)
    mask_pos_f = mask_pos.astype(f32)
    n_assign = jnp.sum(mask_pos_f, axis=0, keepdims=True)

    iou_m = jnp.where(mask_pos, iou, -1.0)
    mxi = jnp.max(iou_m, axis=0, keepdims=True)
    iota_t = jax.lax.broadcasted_iota(jnp.int32, (T, A), 0)
    cand_t = jnp.where(iou_m == mxi, iota_t, T)
    first_t = jnp.min(cand_t, axis=0, keepdims=True)
    is_max = (iota_t == first_t).astype(f32)
    mask_pos_f = jnp.where(n_assign > 1.0, is_max, mask_pos_f)

    metric_pos = metric_m * mask_pos_f
    iou_pos = jnp.where(mask_pos_f > 0.0, iou, 0.0)
    pmm = jnp.max(metric_pos, axis=1, keepdims=True)
    pim = jnp.max(iou_pos, axis=1, keepdims=True)
    q = metric_pos * (pim / (pmm + _EPS))
    q = jnp.maximum(q, mask_pos_f * 1e-30)

    rhs = jnp.concatenate(
        [tb, jnp.ones((T, 1), f32), onehot_tc], axis=1)
    lhs_b = q.astype(bf16)
    rhs0 = rhs.astype(bf16)
    rhs1 = (rhs - rhs0.astype(f32)).astype(bf16)

    def _pkdot(rhs_b):
        return jax.lax.dot_general(
            lhs_b, rhs_b, (((0,), (0,)), ((), ())),
            preferred_element_type=f32)

    packed = _pkdot(rhs0) + _pkdot(rhs1)

    coef_col = packed[:, 4:5]
    fg = coef_col > 0.0
    fg_col = fg.astype(f32)
    denom = jnp.maximum(coef_col, 1e-38)
    bbox_n = jnp.where(fg, packed[:, 0:4] / denom, tb[0:1, :]) * inv_s_col
    align_cls = packed[:, 5:]

    boxes_n = boxes * inv_s_col
    out_ref[0, :, 0:4] = boxes_n
    out_ref[0, :, 4:8] = bbox_n
    out_ref[0, :, 8:9] = fg_col
    out_ref[0, :, 9:] = align_cls


def kernel(cls, boxes, true_classes, true_bboxes):
    B, A, C = cls.shape
    T = true_classes.shape[1]
    anchors, scalers = _anchors_np()
    anc_t = jnp.asarray(anchors.T)
    inv_s = (1.0 / scalers).astype(np.float32)
    inv_s_col = jnp.asarray(inv_s[:, None])
    inv_s_row = jnp.asarray(inv_s[None, :])

    boxes_t = jnp.transpose(boxes, (0, 2, 1))
    tc = true_classes.astype(jnp.int32)[..., None]

    out = pl.pallas_call(
        _yolo_kernel,
        grid=(B,),
        in_specs=[
            pl.BlockSpec((1, A, C), lambda b: (b, 0, 0)),
            pl.BlockSpec((1, A, 4), lambda b: (b, 0, 0)),
            pl.BlockSpec((1, 4, A), lambda b: (b, 0, 0)),
            pl.BlockSpec((1, T, 1), lambda b: (b, 0, 0)),
            pl.BlockSpec((1, T, 4), lambda b: (b, 0, 0)),
            pl.BlockSpec((2, A), lambda b: (0, 0)),
            pl.BlockSpec((A, 1), lambda b: (0, 0)),
            pl.BlockSpec((1, A), lambda b: (0, 0)),
        ],
        out_specs=pl.BlockSpec((1, A, 4 + 4 + 1 + C), lambda b: (b, 0, 0)),
        out_shape=jax.ShapeDtypeStruct((B, A, 4 + 4 + 1 + C), cls.dtype),
        compiler_params=pltpu.CompilerParams(
            dimension_semantics=("arbitrary",),
            vmem_limit_bytes=100 * 1024 * 1024,
        ),
    )(cls, boxes, boxes_t, tc, true_bboxes, anc_t, inv_s_col, inv_s_row)
    return out


# speedup vs baseline: 1.0470x; 1.0353x over previous
import numpy as np
import jax
import jax.numpy as jnp
from jax.experimental import pallas as pl
from jax.experimental.pallas import tpu as pltpu

_FEATURE_MAPS = [(80, 80), (40, 40), (20, 20)]
_INPUT_SIZE = 640.0
_NUM_CLASSES = 80
_TOPK = 10
_CLS_FACTOR = 0.5
_IOU_FACTOR = 6.0
_EPS = 1e-9


def _anchors_np():
    all_a, all_s = [], []
    for (h, w) in _FEATURE_MAPS:
        stride = _INPUT_SIZE / float(h)
        ys = (np.arange(h, dtype=np.float32) + 0.5) * stride
        xs = (np.arange(w, dtype=np.float32) + 0.5) * stride
        gx, gy = np.meshgrid(xs, ys)
        all_a.append(np.stack([gx.reshape(-1), gy.reshape(-1)], axis=-1))
        all_s.append(np.full((h * w,), stride, dtype=np.float32))
    anchors = np.concatenate(all_a, axis=0).astype(np.float32)
    scalers = np.concatenate(all_s, axis=0).astype(np.float32)
    return anchors, scalers


def _yolo_kernel(cls_ref, boxes_ref, boxes_t_ref, tc_ref, tb_ref,
                 anc_t_ref, inv_s_col_ref, inv_s_row_ref, out_ref):
    A = cls_ref.shape[1]
    T = tb_ref.shape[1]
    C = cls_ref.shape[2]
    f32 = jnp.float32

    cls = cls_ref[0]
    boxes = boxes_ref[0]
    boxes_t = boxes_t_ref[0]
    tcls = tc_ref[0]
    tb = tb_ref[0]
    anc_t = anc_t_ref[...]
    inv_s_col = inv_s_col_ref[...]
    inv_s_row = inv_s_row_ref[...]

    iota_c = jax.lax.broadcasted_iota(jnp.int32, (T, C), 1)
    onehot_tc = (iota_c == tcls).astype(f32)
    bf16 = jnp.bfloat16
    oh_b = onehot_tc.astype(bf16)
    p0 = cls.astype(bf16)
    r1 = cls - p0.astype(f32)
    p1 = r1.astype(bf16)
    p2 = (r1 - p1.astype(f32)).astype(bf16)

    def _csdot(rhs_b):
        return jax.lax.dot_general(
            oh_b, rhs_b, (((1,), (1,)), ((), ())),
            preferred_element_type=f32)

    logit = _csdot(p0) + _csdot(p1) + _csdot(p2)
    sqrt_cs = jax.lax.rsqrt(1.0 + jnp.exp(-logit))

    gx1 = tb[:, 0:1]; gy1 = tb[:, 1:2]; gx2 = tb[:, 2:3]; gy2 = tb[:, 3:4]
    px1 = boxes_t[0:1, :]; py1 = boxes_t[1:2, :]
    px2 = boxes_t[2:3, :]; py2 = boxes_t[3:4, :]
    iw = jnp.maximum(jnp.minimum(gx2, px2) - jnp.maximum(gx1, px1), 0.0)
    ih = jnp.maximum(jnp.minimum(gy2, py2) - jnp.maximum(gy1, py1), 0.0)
    inter = iw * ih
    ag = (gx2 - gx1) * (gy2 - gy1)
    ap = (px2 - px1) * (py2 - py1)
    iou = inter / (ag + ap - inter + _EPS)

    iou2 = iou * iou
    metric = sqrt_cs * (iou2 * iou2 * iou2)

    ax = anc_t[0:1, :]; ay = anc_t[1:2, :]
    in_gt = (ax >= gx1) & (ax <= gx2) & (ay >= gy1) & (ay <= gy2)
    metric_m = jnp.where(in_gt, metric, 0.0)

    W = 128
    NC = (A + W - 1) // W
    pad = NC * W - A
    bufs = [jnp.full((T, W), -1.0, f32)] * _TOPK
    for i in range(NC):
        if pad and i == NC - 1:
            c = jnp.concatenate(
                [metric_m[:, i * W:], jnp.zeros((T, pad), f32)], axis=1)
        else:
            c = metric_m[:, i * W:(i + 1) * W]
        for j in range(min(i + 1, _TOPK)):
            hi = jnp.maximum(bufs[j], c)
            if j != _TOPK - 1:
                c = jnp.minimum(bufs[j], c)
            bufs[j] = hi
    buf = jnp.concatenate(bufs, axis=1)
    total = jnp.zeros((T, 1), f32)
    kth = jnp.zeros((T, 1), f32)
    for r in range(_TOPK):
        mx = jnp.max(buf, axis=1, keepdims=True)
        eq = buf == mx
        c = jnp.sum(jnp.where(eq, 1.0, 0.0), axis=1, keepdims=True)
        hit = (total < float(_TOPK)) & (total + c >= float(_TOPK))
        kth = jnp.where(hit, mx, kth)
        total = total + c
        if r != _TOPK - 1:
            buf = jnp.where(eq, -1.0, buf)

    mask_pos = (metric_m >= kth) & (metric_m > 0.0)
    mask_pos_f = mask_pos.astype(f32)
    n_assign = jnp.sum(mask_pos_f, axis=0, keepdims=True)

    iou_m = jnp.where(mask_pos, iou, -1.0)
    mxi = jnp.max(iou_m, axis=0, keepdims=True)
    iota_t = jax.lax.broadcasted_iota(jnp.int32, (T, A), 0)
    cand_t = jnp.where(iou_m == mxi, iota_t, T)
    first_t = jnp.min(cand_t, axis=0, keepdims=True)
    is_max = (iota_t == first_t).astype(f32)
    mask_pos_f = jnp.where(n_assign > 1.0, is_max, mask_pos_f)

    metric_pos = metric_m * mask_pos_f
    iou_pos = jnp.where(mask_pos_f > 0.0, iou, 0.0)
    pmm = jnp.max(metric_pos, axis=1, keepdims=True)
    pim = jnp.max(iou_pos, axis=1, keepdims=True)
    q = metric_pos * (pim / (pmm + _EPS))
    q = jnp.maximum(q, mask_pos_f * 1e-30)

    rhs = jnp.concatenate(
        [tb, jnp.ones((T, 1), f32), onehot_tc], axis=1)
    lhs_b = q.astype(bf16)
    rhs0 = rhs.astype(bf16)
    rhs1 = (rhs - rhs0.astype(f32)).astype(bf16)

    def _pkdot(rhs_b):
        return jax.lax.dot_general(
            lhs_b, rhs_b, (((0,), (0,)), ((), ())),
            preferred_element_type=f32)

    packed = _pkdot(rhs0) + _pkdot(rhs1)

    coef_col = packed[:, 4:5]
    fg = coef_col > 0.0
    fg_col = fg.astype(f32)
    denom = jnp.maximum(coef_col, 1e-38)
    bbox_n = jnp.where(fg, packed[:, 0:4] / denom, tb[0:1, :]) * inv_s_col
    align_cls = packed[:, 5:]

    boxes_n = boxes * inv_s_col
    out_ref[0] = jnp.concatenate([boxes_n, bbox_n, fg_col, align_cls], axis=1)


def kernel(cls, boxes, true_classes, true_bboxes):
    B, A, C = cls.shape
    T = true_classes.shape[1]
    anchors, scalers = _anchors_np()
    anc_t = jnp.asarray(anchors.T)
    inv_s = (1.0 / scalers).astype(np.float32)
    inv_s_col = jnp.asarray(inv_s[:, None])
    inv_s_row = jnp.asarray(inv_s[None, :])

    boxes_t = jnp.transpose(boxes, (0, 2, 1))
    tc = true_classes.astype(jnp.int32)[..., None]

    out = pl.pallas_call(
        _yolo_kernel,
        grid=(B,),
        in_specs=[
            pl.BlockSpec((1, A, C), lambda b: (b, 0, 0)),
            pl.BlockSpec((1, A, 4), lambda b: (b, 0, 0)),
            pl.BlockSpec((1, 4, A), lambda b: (b, 0, 0)),
            pl.BlockSpec((1, T, 1), lambda b: (b, 0, 0)),
            pl.BlockSpec((1, T, 4), lambda b: (b, 0, 0)),
            pl.BlockSpec((2, A), lambda b: (0, 0)),
            pl.BlockSpec((A, 1), lambda b: (0, 0)),
            pl.BlockSpec((1, A), lambda b: (0, 0)),
        ],
        out_specs=pl.BlockSpec((1, A, 4 + 4 + 1 + C), lambda b: (b, 0, 0)),
        out_shape=jax.ShapeDtypeStruct((B, A, 4 + 4 + 1 + C), cls.dtype),
        compiler_params=pltpu.CompilerParams(
            dimension_semantics=("arbitrary",),
            vmem_limit_bytes=100 * 1024 * 1024,
        ),
    )(cls, boxes, boxes_t, tc, true_bboxes, anc_t, inv_s_col, inv_s_row)
    return out
